# Initial kernel scaffold; baseline (speedup 1.0000x reference)
import jax
import jax.numpy as jnp
from jax.experimental import pallas as pl


def kernel(input):
    raise NotImplementedError("write your pallas kernel here")


# speedup vs baseline: 4.9537x; 4.9537x over previous
import functools

import jax
import jax.numpy as jnp
from jax import lax
from jax.experimental import pallas as pl
from jax.experimental.pallas import tpu as pltpu
from jax.experimental.pallas import tpu_sc as plsc

R, C = 128, 32768
L = 16
NV = C // L
NC, NS = 2, 16
NW = NC * NS
RPW = R // NW
NBINS = 256
MINI32 = -2147483648


def _lsr(v, s):
    return lax.shift_right_logical(v, jnp.full(v.shape, s, v.dtype))


def _lsl(v, s):
    return lax.shift_left(v, jnp.full(v.shape, s, v.dtype))


def _key(x):
    b = lax.bitcast_convert_type(x, jnp.int32)
    return jnp.where(b < 0, ~b, b | MINI32)


def _unkey(k):
    b = jnp.where(k < 0, k & jnp.int32(0x7FFFFFFF), ~k)
    return lax.bitcast_convert_type(b, jnp.float32)


def _histo_pass(xbuf, hc, hs, shift, prefix, lvl):
    zi = jnp.zeros((L,), jnp.int32)
    zf = jnp.zeros((L,), jnp.float32)

    def zbody(j, _):
        hc[pl.ds(j * L, L)] = zi
        hs[pl.ds(j * L, L)] = zf
        return 0

    lax.fori_loop(0, NBINS // L, zbody, 0)

    ones = jnp.ones((L,), jnp.int32)

    def body(i, _):
        x = xbuf[pl.ds(i * L, L)]
        k = _key(x)
        dig = _lsr(k, shift) & 255
        if lvl == 0:
            mask = None
        else:
            mask = _lsr(k, shift + 8) == prefix
        plsc.addupdate_scatter(hc, [dig], ones, mask=mask)
        plsc.addupdate_scatter(hs, [dig], x, mask=mask)
        return 0

    lax.fori_loop(0, NV, body, 0)


def _scan_level(hc, hs, shift, prefix, k_acc, s_acc):
    iota = lax.iota(jnp.int32, L)
    k_acc_f = k_acc.astype(jnp.float32)

    def body(jj, carry):
        b_sel, kx, sx, cb, sb, k_run, s_run = carry
        j = (NBINS // L - 1) - jj
        c_v = hc[pl.ds(j * L, L)]
        s_v = hs[pl.ds(j * L, L)]
        c_d = lax.rev(c_v, (0,))
        s_d = lax.rev(s_v, (0,))
        d_vec = j * L + (L - 1) - iota
        sufK = plsc.cumsum(c_d) + k_run
        sufS = plsc.cumsum(s_d) + s_run
        kedge = _lsl((prefix << 8) | d_vec, shift)
        e = _unkey(kedge)
        f = (s_acc + sufS) - (k_acc_f + sufK.astype(jnp.float32)) * e - 1.0
        cond = f >= 0.0
        cand = jnp.where(cond, d_vec, -1)
        lsel = jnp.max(cand)
        upd = lsel > b_sel
        lm = cond & (d_vec == lsel)
        kx_n = jnp.max(jnp.where(lm, sufK, MINI32))
        sx_n = jnp.max(jnp.where(lm, sufS, -jnp.inf))
        cb_n = jnp.max(jnp.where(lm, c_d, MINI32))
        sb_n = jnp.max(jnp.where(lm, s_d, -jnp.inf))
        b_sel = jnp.where(upd, lsel, b_sel)
        kx = jnp.where(upd, kx_n, kx)
        sx = jnp.where(upd, sx_n, sx)
        cb = jnp.where(upd, cb_n, cb)
        sb = jnp.where(upd, sb_n, sb)
        k_run = k_run + jnp.sum(c_d)
        s_run = s_run + jnp.sum(s_d)
        return b_sel, kx, sx, cb, sb, k_run, s_run

    init = (jnp.int32(-1), jnp.int32(0), jnp.float32(0.0), jnp.int32(0),
            jnp.float32(0.0), jnp.int32(0), jnp.float32(0.0))
    b_sel, kx, sx, cb, sb, _, _ = lax.fori_loop(0, NBINS // L, body, init)
    prefix = (prefix << 8) | b_sel
    k_acc = k_acc + (kx - cb)
    s_acc = s_acc + (sx - sb)
    return prefix, k_acc, s_acc


@functools.lru_cache(maxsize=1)
def _build():
    mesh = plsc.VectorSubcoreMesh(core_axis_name="c", subcore_axis_name="s",
                                  num_cores=NC, num_subcores=NS)

    @functools.partial(
        pl.kernel,
        out_type=jax.ShapeDtypeStruct((R, C), jnp.float32),
        mesh=mesh,
        compiler_params=pltpu.CompilerParams(needs_layout_passes=False),
        scratch_types=[
            pltpu.VMEM((C,), jnp.float32),
            pltpu.VMEM((C,), jnp.float32),
            pltpu.VMEM((NBINS,), jnp.int32),
            pltpu.VMEM((NBINS,), jnp.float32),
        ],
    )
    def _sparsemax_sc(in_hbm, out_hbm, xbuf, obuf, hc, hs):
        wid = lax.axis_index("s") * NC + lax.axis_index("c")

        def row_body(r, _):
            row = wid * RPW + r
            pltpu.sync_copy(in_hbm.at[row], xbuf)

            prefix = jnp.int32(0)
            k_acc = jnp.int32(0)
            s_acc = jnp.float32(0.0)
            for lvl, shift in enumerate((24, 16, 8, 0)):
                _histo_pass(xbuf, hc, hs, shift, prefix, lvl)
                prefix, k_acc, s_acc = _scan_level(hc, hs, shift, prefix,
                                                   k_acc, s_acc)
            tau = (jnp.full((L,), s_acc - 1.0, jnp.float32)
                   / jnp.full((L,), k_acc, jnp.int32).astype(jnp.float32))

            def obody(i, _):
                x = xbuf[pl.ds(i * L, L)]
                obuf[pl.ds(i * L, L)] = jnp.maximum(x - tau, 0.0)
                return 0

            lax.fori_loop(0, NV, obody, 0)
            pltpu.sync_copy(obuf, out_hbm.at[row])
            return 0

        lax.fori_loop(0, RPW, row_body, 0)

    return _sparsemax_sc


def kernel(input):
    return _build()(input)


# speedup vs baseline: 14.0693x; 2.8402x over previous
import functools

import jax
import jax.numpy as jnp
from jax import lax
from jax.experimental import pallas as pl
from jax.experimental.pallas import tpu as pltpu
from jax.experimental.pallas import tpu_sc as plsc

R, C = 128, 32768
L = 16
NV = C // L
NC, NS = 2, 16
NW = NC * NS
RPW = R // NW
NBINS = 256
MINI32 = -2147483648


def _lsr(v, s):
    return lax.shift_right_logical(v, jnp.full(v.shape, s, v.dtype))


def _lsl(v, s):
    return lax.shift_left(v, jnp.full(v.shape, s, v.dtype))


def _key(x):
    b = lax.bitcast_convert_type(x, jnp.int32)
    return jnp.where(b < 0, ~b, b | MINI32)


def _unkey(k):
    b = jnp.where(k < 0, k & jnp.int32(0x7FFFFFFF), ~k)
    return lax.bitcast_convert_type(b, jnp.float32)


def _histo_pass(xbuf, hc, hs, shift, prefix, lvl):
    zi = jnp.zeros((L,), jnp.int32)
    zf = jnp.zeros((L,), jnp.float32)

    @plsc.parallel_loop(0, NBINS // L, unroll=NBINS // L)
    def _(j):
        hc[pl.ds(j * L, L)] = zi
        hs[pl.ds(j * L, L)] = zf

    ones = jnp.ones((L,), jnp.int32)

    @plsc.parallel_loop(0, NV, unroll=8)
    def _(i):
        x = xbuf[pl.ds(i * L, L)]
        k = _key(x)
        dig = _lsr(k, shift) & 255
        if lvl == 0:
            mask = None
        else:
            mask = _lsr(k, shift + 8) == prefix
        plsc.addupdate_scatter(hc, [dig], ones, mask=mask)
        plsc.addupdate_scatter(hs, [dig], x, mask=mask)


def _fast_scan(hc, hs, shift, prefix, k_acc, s_acc):
    iota = lax.iota(jnp.int32, L)
    k_acc_f = k_acc.astype(jnp.float32)

    @plsc.parallel_loop(0, L, unroll=L, carry=(jnp.zeros((L,), jnp.int32),
                                               jnp.zeros((L,), jnp.float32)))
    def _gtot(jj, carry):
        tc, ts = carry
        idx = iota * L + jj
        tc = tc + plsc.load_gather(hc, [idx])
        ts = ts + plsc.load_gather(hs, [idx])
        return tc, ts

    tot_c, tot_s = _gtot

    blk_desc = (L - 1) - iota
    rc = lax.rev(tot_c, (0,))
    rs = lax.rev(tot_s, (0,))
    sufKb = plsc.cumsum(rc)
    sufSb = plsc.cumsum(rs)
    e_b = _unkey(_lsl((prefix << 8) | (blk_desc * L), shift))
    f_b = ((s_acc + sufSb)
           - (k_acc_f + sufKb.astype(jnp.float32)) * e_b - 1.0)
    condb = f_b >= 0.0
    bsel = jnp.max(jnp.where(condb, blk_desc, -1))
    lmb = condb & (blk_desc == bsel)
    k_abv = jnp.max(jnp.where(lmb, sufKb - rc, MINI32))
    s_abv = jnp.max(jnp.where(lmb, sufSb - rs, -jnp.inf))

    c_v = hc[pl.ds(bsel * L, L)]
    s_v = hs[pl.ds(bsel * L, L)]
    c_d = lax.rev(c_v, (0,))
    s_d = lax.rev(s_v, (0,))
    d_vec = bsel * L + (L - 1) - iota
    sufK = plsc.cumsum(c_d) + k_abv
    sufS = plsc.cumsum(s_d) + s_abv
    e = _unkey(_lsl((prefix << 8) | d_vec, shift))
    f = (s_acc + sufS) - (k_acc_f + sufK.astype(jnp.float32)) * e - 1.0
    cond = f >= 0.0
    cand = jnp.where(cond, d_vec, -1)
    lsel = jnp.max(cand)
    lm = cond & (d_vec == lsel)
    kx = jnp.max(jnp.where(lm, sufK, MINI32))
    sx = jnp.max(jnp.where(lm, sufS, -jnp.inf))
    cb = jnp.max(jnp.where(lm, c_d, MINI32))
    sb = jnp.max(jnp.where(lm, s_d, -jnp.inf))

    prefix = (prefix << 8) | lsel
    k_acc = k_acc + (kx - cb)
    s_acc = s_acc + (sx - sb)
    return prefix, k_acc, s_acc


@functools.lru_cache(maxsize=1)
def _build():
    mesh = plsc.VectorSubcoreMesh(core_axis_name="c", subcore_axis_name="s",
                                  num_cores=NC, num_subcores=NS)

    @functools.partial(
        pl.kernel,
        out_type=jax.ShapeDtypeStruct((R, C), jnp.float32),
        mesh=mesh,
        compiler_params=pltpu.CompilerParams(needs_layout_passes=False),
        scratch_types=[
            pltpu.VMEM((C,), jnp.float32),
            pltpu.VMEM((C,), jnp.float32),
            pltpu.VMEM((C,), jnp.float32),
            pltpu.VMEM((NBINS,), jnp.int32),
            pltpu.VMEM((NBINS,), jnp.float32),
            pltpu.SemaphoreType.DMA,
        ],
    )
    def _sparsemax_sc(in_hbm, out_hbm, xb0, xb1, ob, hc, hs, in_sem):
        wid = lax.axis_index("s") * NC + lax.axis_index("c")
        base = wid * RPW
        xbufs = (xb0, xb1)

        pltpu.async_copy(in_hbm.at[base], xb0, in_sem)
        for r in range(RPW):
            xb = xbufs[r % 2]
            pltpu.make_async_copy(in_hbm.at[base + r], xb, in_sem).wait()
            if r + 1 < RPW:
                pltpu.async_copy(in_hbm.at[base + r + 1], xbufs[(r + 1) % 2],
                                 in_sem)

            prefix = jnp.int32(0)
            k_acc = jnp.int32(0)
            s_acc = jnp.float32(0.0)
            for lvl, shift in enumerate((24, 16, 8, 0)):
                _histo_pass(xb, hc, hs, shift, prefix, lvl)
                prefix, k_acc, s_acc = _fast_scan(hc, hs, shift, prefix,
                                                  k_acc, s_acc)
            tau = (jnp.full((L,), s_acc - 1.0, jnp.float32)
                   / jnp.full((L,), k_acc, jnp.int32).astype(jnp.float32))

            @plsc.parallel_loop(0, NV, unroll=8)
            def _(i):
                x = xb[pl.ds(i * L, L)]
                ob[pl.ds(i * L, L)] = jnp.maximum(x - tau, 0.0)
            pltpu.sync_copy(ob, out_hbm.at[base + r])

    return _sparsemax_sc


def kernel(input):
    return _build()(input)
